# Initial kernel scaffold; baseline (speedup 1.0000x reference)
import jax
import jax.numpy as jnp
from jax.experimental import pallas as pl


def kernel(x, edge_feature, edge_index, node_positions, params):
    raise NotImplementedError("write your pallas kernel here")


# speedup vs baseline: 1.0055x; 1.0055x over previous
import functools

import jax
import jax.numpy as jnp
import numpy as np
from jax.experimental import pallas as pl
from jax.experimental.pallas import tpu as pltpu

_H = 8
_DNP = 16
_DEP = 16
_TEMP = float(np.sqrt(_DEP))


def _dot(a, b):
    return jnp.dot(a, b, preferred_element_type=jnp.float32)


def _edge_body(xi, ef, rev, xj, dfeat,
               w1a, w1b, w1c, w1d, b1, w2, b2,
               wxa, wea, ba, wb2, bb2,
               wv, bv, wd1, bd1, wd2, bd2,
               ue_out, msg_out):
    xi_ = xi[...]
    ef_ = ef[...]
    rev_ = rev[...]
    xj_ = xj[...]

    h1 = (_dot(xi_, w1a[...]) + _dot(ef_, w1b[...]) +
          _dot(rev_, w1c[...]) + _dot(xj_, w1d[...]) + b1[...])
    h1 = jnp.maximum(h1, 0.0)
    ue_out[...] = _dot(h1, w2[...]) + b2[...]

    ah = jnp.maximum(_dot(xi_, wxa[...]) + _dot(ef_, wea[...]) + ba[...], 0.0)
    att = _dot(ah, wb2[...]) + bb2[...]
    s = att * (1.0 / _TEMP)
    m = s
    for sh in (64, 32, 16):
        m = jnp.maximum(m, pltpu.roll(m, sh, 1))
    e = jnp.exp(s - m)
    z = e
    for sh in (64, 32, 16):
        z = z + pltpu.roll(z, sh, 1)
    prob = e / z

    d1 = jnp.maximum(_dot(dfeat[...], wd1[...]) + bd1[...], 0.0)
    d2 = _dot(d1, wd2[...]) + bd2[...]
    dm = 1.0 / (1.0 + jnp.exp(-d2))

    v = _dot(xj_, wv[...]) + bv[...]
    msg_out[...] = prob * dm * v


def _node_body(x, agg, so, si, dgo, dgi,
               weo, wei, be, wn1a, wn1b, bn1, wn2, bn2,
               wm1, bm1, wm2, bm2, out):
    deg_o = dgo[...]
    deg_i = dgi[...]
    om = so[...] / jnp.maximum(deg_o, 1.0)
    im = si[...] / jnp.maximum(deg_i, 1.0)
    ctx = _dot(om, weo[...]) + _dot(im, wei[...]) + be[...]
    aggm = jnp.where(deg_o > 0.0, agg[...], 0.0)
    h = jnp.maximum(_dot(aggm, wn1a[...]) + _dot(ctx, wn1b[...]) + bn1[...], 0.0)
    nu = _dot(h, wn2[...]) + bn2[...]
    nl = jnp.maximum(_dot(nu, wm1[...]) + bm1[...], 0.0)
    out[...] = x[...] + _dot(nl, wm2[...]) + bm2[...]


def _full(shape):
    return pl.BlockSpec(shape, lambda i: (0,) * len(shape))


def kernel(x, edge_feature, edge_index, node_positions, params):
    f32 = jnp.float32
    n_nodes, d_node = x.shape
    n_edges, d_edge = edge_feature.shape
    row = edge_index[0].astype(jnp.int32)
    col = edge_index[1].astype(jnp.int32)

    keys = row * n_nodes + col
    rkeys = col * n_nodes + row
    order = jnp.argsort(keys)
    skeys = keys[order]
    pos = jnp.searchsorted(skeys, rkeys)
    pos_c = jnp.clip(pos, 0, n_edges - 1)
    found = skeys[pos_c] == rkeys
    gidx = order[pos_c]

    xi = jnp.take(x, row, axis=0)
    xj = jnp.take(x, col, axis=0)
    rev = jnp.where(found[:, None], jnp.take(edge_feature, gidx, axis=0), 0.0)
    diff = jnp.take(node_positions, row, axis=0) - jnp.take(node_positions, col, axis=0)
    dist = jnp.sqrt(jnp.sum(diff * diff, axis=-1, keepdims=True))
    dfeat = jnp.concatenate([diff, dist], axis=-1)

    (w1, b1), (w2, b2) = params['nn_edge_update']
    w1a, w1b, w1c, w1d = w1[:128], w1[128:256], w1[256:384], w1[384:512]
    (wq, bq) = params['proj_q'][0]
    (wk, bk) = params['proj_k'][0]
    (wv, bv) = params['proj_v'][0]
    (wa1, ba1), (wa2, ba2) = params['nn_att']
    bq_ = jnp.zeros((128, 256), f32)
    bk_ = jnp.zeros((128, 256), f32)
    b2_ = jnp.zeros((256, 128), f32)
    for h in range(_H):
        bq_ = bq_.at[16 * h:16 * h + 16, 32 * h:32 * h + 32].set(wa1[:16])
        bk_ = bk_.at[16 * h:16 * h + 16, 32 * h:32 * h + 32].set(wa1[16:])
        b2_ = b2_.at[32 * h:32 * h + 32, 16 * h:16 * h + 16].set(wa2)
    wxa = wq @ bq_
    wea = wk @ bk_
    ba = (bq @ bq_ + bk @ bk_ + jnp.tile(ba1, _H))[None]
    wb2 = b2_
    bb2 = jnp.tile(ba2, _H)[None]
    (wd1, bd1), (wd2, bd2) = params['distance_mlp']

    be_blk = min(1000, n_edges)
    grid_e = n_edges // be_blk
    espec = pl.BlockSpec((be_blk, d_node), lambda i: (i, 0))
    ue, msg = pl.pallas_call(
        _edge_body,
        grid=(grid_e,),
        in_specs=[espec, espec, espec, espec,
                  pl.BlockSpec((be_blk, 4), lambda i: (i, 0)),
                  _full((128, 384)), _full((128, 384)), _full((128, 384)),
                  _full((128, 384)), _full((1, 384)),
                  _full((384, 128)), _full((1, 128)),
                  _full((128, 256)), _full((128, 256)), _full((1, 256)),
                  _full((256, 128)), _full((1, 128)),
                  _full((128, 128)), _full((1, 128)),
                  _full((4, 32)), _full((1, 32)),
                  _full((32, 1)), _full((1, 1))],
        out_specs=[espec, espec],
        out_shape=[jax.ShapeDtypeStruct((n_edges, d_edge), f32),
                   jax.ShapeDtypeStruct((n_edges, d_node), f32)],
        compiler_params=pltpu.CompilerParams(
            dimension_semantics=("arbitrary",)),
    )(xi, edge_feature, rev, xj, dfeat,
      w1a, w1b, w1c, w1d, b1[None], w2, b2[None],
      wxa, wea, ba, wb2, bb2,
      wv, bv[None], wd1, bd1[None], wd2, bd2[None])

    ones = jnp.ones((n_edges,), f32)
    deg_out = jax.ops.segment_sum(ones, row, num_segments=n_nodes)
    deg_in = jax.ops.segment_sum(ones, col, num_segments=n_nodes)
    agg = jax.ops.segment_max(msg, row, num_segments=n_nodes)
    agg = jnp.where(deg_out[:, None] > 0, agg, 0.0)
    s_out = jax.ops.segment_sum(ue, row, num_segments=n_nodes)
    s_in = jax.ops.segment_sum(ue, col, num_segments=n_nodes)

    we_w, we_b = params['edge_attention_mlp'][0]
    weo, wei = we_w[:128], we_w[128:]
    (wn1, bn1), (wn2, bn2) = params['nn_node_update']
    wn1a, wn1b = wn1[:128], wn1[128:]
    (wm1, bm1), (wm2, bm2) = params['node_nonlinear_mlp']

    bn_blk = min(1000, n_nodes)
    grid_n = n_nodes // bn_blk
    nspec = pl.BlockSpec((bn_blk, d_node), lambda i: (i, 0))
    dspec = pl.BlockSpec((bn_blk, 1), lambda i: (i, 0))
    new_node = pl.pallas_call(
        _node_body,
        grid=(grid_n,),
        in_specs=[nspec, nspec, nspec, nspec, dspec, dspec,
                  _full((128, 128)), _full((128, 128)), _full((1, 128)),
                  _full((128, 256)), _full((128, 256)), _full((1, 256)),
                  _full((256, 128)), _full((1, 128)),
                  _full((128, 128)), _full((1, 128)),
                  _full((128, 128)), _full((1, 128))],
        out_specs=nspec,
        out_shape=jax.ShapeDtypeStruct((n_nodes, d_node), f32),
        compiler_params=pltpu.CompilerParams(
            dimension_semantics=("arbitrary",)),
    )(x, agg, s_out, s_in, deg_out[:, None], deg_in[:, None],
      weo, wei, we_b[None], wn1a, wn1b, bn1[None], wn2, bn2[None],
      wm1, bm1[None], wm2, bm2[None])

    return new_node, ue


# speedup vs baseline: 1.1871x; 1.1806x over previous
import functools

import jax
import jax.numpy as jnp
import numpy as np
from jax import lax
from jax.experimental import pallas as pl
from jax.experimental.pallas import tpu as pltpu
from jax.experimental.pallas import tpu_sc as plsc

_H = 8
_DNP = 16
_DEP = 16
_TEMP = float(np.sqrt(_DEP))


def _dot(a, b):
    return jnp.dot(a, b, preferred_element_type=jnp.float32)


_NW = 32
_GCH = 400


def _sc_gathers(x, ef, row, col, gidx):
    n_edges = row.shape[0]
    d = x.shape[1]
    epw = n_edges // _NW
    gnc = epw // _GCH
    f32 = jnp.float32

    idx3 = (jnp.stack([row, col, gidx])
            .reshape(3, _NW, gnc, _GCH)
            .transpose(1, 0, 2, 3)
            .reshape(_NW, 3 * gnc, _GCH))

    mesh = plsc.VectorSubcoreMesh(core_axis_name="c", subcore_axis_name="s")

    @functools.partial(
        pl.kernel,
        out_type=[jax.ShapeDtypeStruct((n_edges, d), f32),
                  jax.ShapeDtypeStruct((n_edges, d), f32),
                  jax.ShapeDtypeStruct((n_edges, d), f32)],
        mesh=mesh,
        scratch_types=[pltpu.VMEM((_GCH,), jnp.int32),
                       pltpu.VMEM((_GCH, d), f32),
                       pltpu.SemaphoreType.DMA],
    )
    def gather_kernel(x_hbm, ef_hbm, idx3_hbm,
                      xi_hbm, xj_hbm, rev_hbm,
                      idx_c, rbuf, sem):
        wid = lax.axis_index("s") * 2 + lax.axis_index("c")
        base = wid * epw

        def chunk(c, carry):
            off = base + c * _GCH
            sl = pl.ds(off, _GCH)
            pltpu.sync_copy(idx3_hbm.at[wid, c], idx_c)
            pltpu.async_copy(x_hbm.at[idx_c], rbuf, sem).wait()
            pltpu.sync_copy(rbuf, xi_hbm.at[sl])
            pltpu.sync_copy(idx3_hbm.at[wid, gnc + c], idx_c)
            pltpu.async_copy(x_hbm.at[idx_c], rbuf, sem).wait()
            pltpu.sync_copy(rbuf, xj_hbm.at[sl])
            pltpu.sync_copy(idx3_hbm.at[wid, 2 * gnc + c], idx_c)
            pltpu.async_copy(ef_hbm.at[idx_c], rbuf, sem).wait()
            pltpu.sync_copy(rbuf, rev_hbm.at[sl])
            return carry

        lax.fori_loop(0, gnc, chunk, 0)

    return gather_kernel(x, ef, idx3)


def _edge_body(xi, ef, rev, xj, pi, pj, foundf,
               w1a, w1b, w1c, w1d, b1, w2, b2,
               wxa, wea, ba, wb2, bb2,
               wv, bv, wd1, bd1, wd2, bd2,
               ue_out, msg_out):
    xi_ = xi[...]
    ef_ = ef[...]
    rev_ = rev[...] * foundf[...]
    xj_ = xj[...]
    diff = pi[...] - pj[...]
    dist = jnp.sqrt(jnp.sum(diff * diff, axis=1, keepdims=True))
    lane = lax.broadcasted_iota(jnp.int32, diff.shape, 1)
    dfeat = diff + jnp.where(lane == 3, dist, 0.0)

    h1 = (_dot(xi_, w1a[...]) + _dot(ef_, w1b[...]) +
          _dot(rev_, w1c[...]) + _dot(xj_, w1d[...]) + b1[...])
    h1 = jnp.maximum(h1, 0.0)
    ue_out[...] = _dot(h1, w2[...]) + b2[...]

    ah = jnp.maximum(_dot(xi_, wxa[...]) + _dot(ef_, wea[...]) + ba[...], 0.0)
    att = _dot(ah, wb2[...]) + bb2[...]
    s = att * (1.0 / _TEMP)
    m = s
    for sh in (64, 32, 16):
        m = jnp.maximum(m, pltpu.roll(m, sh, 1))
    e = jnp.exp(s - m)
    z = e
    for sh in (64, 32, 16):
        z = z + pltpu.roll(z, sh, 1)
    prob = e / z

    d1 = jnp.maximum(_dot(dfeat, wd1[...]) + bd1[...], 0.0)
    d2 = _dot(d1, wd2[...]) + bd2[...]
    dm = 1.0 / (1.0 + jnp.exp(-d2))

    v = _dot(xj_, wv[...]) + bv[...]
    msg_out[...] = prob * dm * v


def _node_body(x, agg, so, si, dgo, dgi,
               weo, wei, be, wn1a, wn1b, bn1, wn2, bn2,
               wm1, bm1, wm2, bm2, out):
    deg_o = dgo[...]
    deg_i = dgi[...]
    om = so[...] / jnp.maximum(deg_o, 1.0)
    im = si[...] / jnp.maximum(deg_i, 1.0)
    ctx = _dot(om, weo[...]) + _dot(im, wei[...]) + be[...]
    aggm = jnp.where(deg_o > 0.0, agg[...], 0.0)
    h = jnp.maximum(_dot(aggm, wn1a[...]) + _dot(ctx, wn1b[...]) + bn1[...], 0.0)
    nu = _dot(h, wn2[...]) + bn2[...]
    nl = jnp.maximum(_dot(nu, wm1[...]) + bm1[...], 0.0)
    out[...] = x[...] + _dot(nl, wm2[...]) + bm2[...]


def _full(shape):
    return pl.BlockSpec(shape, lambda i: (0,) * len(shape))


def kernel(x, edge_feature, edge_index, node_positions, params):
    f32 = jnp.float32
    n_nodes, d_node = x.shape
    n_edges, d_edge = edge_feature.shape
    row = edge_index[0].astype(jnp.int32)
    col = edge_index[1].astype(jnp.int32)

    keys = row * n_nodes + col
    rkeys = col * n_nodes + row
    order = jnp.argsort(keys)
    skeys = keys[order]
    pos = jnp.searchsorted(skeys, rkeys)
    pos_c = jnp.clip(pos, 0, n_edges - 1)
    found = skeys[pos_c] == rkeys
    gidx = order[pos_c]

    posp = jnp.pad(node_positions, ((0, 0), (0, 1)))
    xi, xj, rev = _sc_gathers(x, edge_feature, row, col, gidx)
    pi = jnp.take(posp, row, axis=0)
    pj = jnp.take(posp, col, axis=0)
    foundf = found.astype(f32)[:, None]

    (w1, b1), (w2, b2) = params['nn_edge_update']
    w1a, w1b, w1c, w1d = w1[:128], w1[128:256], w1[256:384], w1[384:512]
    (wq, bq) = params['proj_q'][0]
    (wk, bk) = params['proj_k'][0]
    (wv, bv) = params['proj_v'][0]
    (wa1, ba1), (wa2, ba2) = params['nn_att']
    bq_ = jnp.zeros((128, 256), f32)
    bk_ = jnp.zeros((128, 256), f32)
    b2_ = jnp.zeros((256, 128), f32)
    for h in range(_H):
        bq_ = bq_.at[16 * h:16 * h + 16, 32 * h:32 * h + 32].set(wa1[:16])
        bk_ = bk_.at[16 * h:16 * h + 16, 32 * h:32 * h + 32].set(wa1[16:])
        b2_ = b2_.at[32 * h:32 * h + 32, 16 * h:16 * h + 16].set(wa2)
    wxa = wq @ bq_
    wea = wk @ bk_
    ba = (bq @ bq_ + bk @ bk_ + jnp.tile(ba1, _H))[None]
    wb2 = b2_
    bb2 = jnp.tile(ba2, _H)[None]
    (wd1, bd1), (wd2, bd2) = params['distance_mlp']

    be_blk = min(1000, n_edges)
    grid_e = n_edges // be_blk
    espec = pl.BlockSpec((be_blk, d_node), lambda i: (i, 0))
    pspec = pl.BlockSpec((be_blk, 4), lambda i: (i, 0))
    fspec = pl.BlockSpec((be_blk, 1), lambda i: (i, 0))
    ue, msg = pl.pallas_call(
        _edge_body,
        grid=(grid_e,),
        in_specs=[espec, espec, espec, espec, pspec, pspec, fspec,
                  _full((128, 384)), _full((128, 384)), _full((128, 384)),
                  _full((128, 384)), _full((1, 384)),
                  _full((384, 128)), _full((1, 128)),
                  _full((128, 256)), _full((128, 256)), _full((1, 256)),
                  _full((256, 128)), _full((1, 128)),
                  _full((128, 128)), _full((1, 128)),
                  _full((4, 32)), _full((1, 32)),
                  _full((32, 1)), _full((1, 1))],
        out_specs=[espec, espec],
        out_shape=[jax.ShapeDtypeStruct((n_edges, d_edge), f32),
                   jax.ShapeDtypeStruct((n_edges, d_node), f32)],
        compiler_params=pltpu.CompilerParams(
            dimension_semantics=("arbitrary",)),
    )(xi, edge_feature, rev, xj, pi, pj, foundf,
      w1a, w1b, w1c, w1d, b1[None], w2, b2[None],
      wxa, wea, ba, wb2, bb2,
      wv, bv[None], wd1, bd1[None], wd2, bd2[None])

    ones = jnp.ones((n_edges,), f32)
    deg_out = jax.ops.segment_sum(ones, row, num_segments=n_nodes)
    deg_in = jax.ops.segment_sum(ones, col, num_segments=n_nodes)
    agg = jax.ops.segment_max(msg, row, num_segments=n_nodes)
    agg = jnp.where(deg_out[:, None] > 0, agg, 0.0)
    s_out = jax.ops.segment_sum(ue, row, num_segments=n_nodes)
    s_in = jax.ops.segment_sum(ue, col, num_segments=n_nodes)

    we_w, we_b = params['edge_attention_mlp'][0]
    weo, wei = we_w[:128], we_w[128:]
    (wn1, bn1), (wn2, bn2) = params['nn_node_update']
    wn1a, wn1b = wn1[:128], wn1[128:]
    (wm1, bm1), (wm2, bm2) = params['node_nonlinear_mlp']

    bn_blk = min(1000, n_nodes)
    grid_n = n_nodes // bn_blk
    nspec = pl.BlockSpec((bn_blk, d_node), lambda i: (i, 0))
    dspec = pl.BlockSpec((bn_blk, 1), lambda i: (i, 0))
    new_node = pl.pallas_call(
        _node_body,
        grid=(grid_n,),
        in_specs=[nspec, nspec, nspec, nspec, dspec, dspec,
                  _full((128, 128)), _full((128, 128)), _full((1, 128)),
                  _full((128, 256)), _full((128, 256)), _full((1, 256)),
                  _full((256, 128)), _full((1, 128)),
                  _full((128, 128)), _full((1, 128)),
                  _full((128, 128)), _full((1, 128))],
        out_specs=nspec,
        out_shape=jax.ShapeDtypeStruct((n_nodes, d_node), f32),
        compiler_params=pltpu.CompilerParams(
            dimension_semantics=("arbitrary",)),
    )(x, agg, s_out, s_in, deg_out[:, None], deg_in[:, None],
      weo, wei, we_b[None], wn1a, wn1b, bn1[None], wn2, bn2[None],
      wm1, bm1[None], wm2, bm2[None])

    return new_node, ue


# speedup vs baseline: 1.2122x; 1.0212x over previous
import functools

import jax
import jax.numpy as jnp
import numpy as np
from jax import lax
from jax.experimental import pallas as pl
from jax.experimental.pallas import tpu as pltpu
from jax.experimental.pallas import tpu_sc as plsc

_H = 8
_DNP = 16
_DEP = 16
_TEMP = float(np.sqrt(_DEP))


def _dot(a, b):
    return jnp.dot(a, b, preferred_element_type=jnp.float32)


_NW = 32
_GCH = 400


def _sc_gathers(x, ef, row, col, gidx):
    n_edges = row.shape[0]
    d = x.shape[1]
    epw = n_edges // _NW
    gnc = epw // _GCH
    f32 = jnp.float32

    idx3 = (jnp.stack([row, col, gidx])
            .reshape(3, _NW, gnc, _GCH)
            .transpose(1, 0, 2, 3)
            .reshape(_NW, 3 * gnc, _GCH))

    mesh = plsc.VectorSubcoreMesh(core_axis_name="c", subcore_axis_name="s")

    @functools.partial(
        pl.kernel,
        out_type=[jax.ShapeDtypeStruct((n_edges, d), f32),
                  jax.ShapeDtypeStruct((n_edges, d), f32),
                  jax.ShapeDtypeStruct((n_edges, d), f32)],
        mesh=mesh,
        scratch_types=[pltpu.VMEM((_GCH,), jnp.int32),
                       pltpu.VMEM((_GCH, d), f32),
                       pltpu.SemaphoreType.DMA],
    )
    def gather_kernel(x_hbm, ef_hbm, idx3_hbm,
                      xi_hbm, xj_hbm, rev_hbm,
                      idx_c, rbuf, sem):
        wid = lax.axis_index("s") * 2 + lax.axis_index("c")
        base = wid * epw

        def chunk(c, carry):
            off = base + c * _GCH
            sl = pl.ds(off, _GCH)
            pltpu.sync_copy(idx3_hbm.at[wid, c], idx_c)
            pltpu.async_copy(x_hbm.at[idx_c], rbuf, sem).wait()
            pltpu.sync_copy(rbuf, xi_hbm.at[sl])
            pltpu.sync_copy(idx3_hbm.at[wid, gnc + c], idx_c)
            pltpu.async_copy(x_hbm.at[idx_c], rbuf, sem).wait()
            pltpu.sync_copy(rbuf, xj_hbm.at[sl])
            pltpu.sync_copy(idx3_hbm.at[wid, 2 * gnc + c], idx_c)
            pltpu.async_copy(ef_hbm.at[idx_c], rbuf, sem).wait()
            pltpu.sync_copy(rbuf, rev_hbm.at[sl])
            return carry

        lax.fori_loop(0, gnc, chunk, 0)

    return gather_kernel(x, ef, idx3)


def _edge_body(xi, ef, rev, xj, pi, pj, foundf,
               w1a, w1b, w1c, w1d, b1, w2, b2,
               wxa, wea, ba, wb2, bb2,
               wv, bv, wd1, bd1, wd2, bd2,
               ue_out, msg_out):
    xi_ = xi[...]
    ef_ = ef[...]
    rev_ = rev[...] * foundf[...]
    xj_ = xj[...]
    diff = pi[...] - pj[...]
    dist = jnp.sqrt(jnp.sum(diff * diff, axis=1, keepdims=True))
    lane = lax.broadcasted_iota(jnp.int32, diff.shape, 1)
    dfeat = diff + jnp.where(lane == 3, dist, 0.0)

    h1 = (_dot(xi_, w1a[...]) + _dot(ef_, w1b[...]) +
          _dot(rev_, w1c[...]) + _dot(xj_, w1d[...]) + b1[...])
    h1 = jnp.maximum(h1, 0.0)
    ue_out[...] = _dot(h1, w2[...]) + b2[...]

    ah = jnp.maximum(_dot(xi_, wxa[...]) + _dot(ef_, wea[...]) + ba[...], 0.0)
    att = _dot(ah, wb2[...]) + bb2[...]
    s = att * (1.0 / _TEMP)
    m = s
    for sh in (64, 32, 16):
        m = jnp.maximum(m, pltpu.roll(m, sh, 1))
    e = jnp.exp(s - m)
    z = e
    for sh in (64, 32, 16):
        z = z + pltpu.roll(z, sh, 1)
    prob = e / z

    d1 = jnp.maximum(_dot(dfeat, wd1[...]) + bd1[...], 0.0)
    d2 = _dot(d1, wd2[...]) + bd2[...]
    dm = 1.0 / (1.0 + jnp.exp(-d2))

    v = _dot(xj_, wv[...]) + bv[...]
    msg_out[...] = prob * dm * v


def _node_body(x, agg, so, si, dgo, dgi,
               weo, wei, be, wn1a, wn1b, bn1, wn2, bn2,
               wm1, bm1, wm2, bm2, out):
    deg_o = dgo[...]
    deg_i = dgi[...]
    om = so[...] / jnp.maximum(deg_o, 1.0)
    im = si[...] / jnp.maximum(deg_i, 1.0)
    ctx = _dot(om, weo[...]) + _dot(im, wei[...]) + be[...]
    aggm = jnp.where(deg_o > 0.0, agg[...], 0.0)
    h = jnp.maximum(_dot(aggm, wn1a[...]) + _dot(ctx, wn1b[...]) + bn1[...], 0.0)
    nu = _dot(h, wn2[...]) + bn2[...]
    nl = jnp.maximum(_dot(nu, wm1[...]) + bm1[...], 0.0)
    out[...] = x[...] + _dot(nl, wm2[...]) + bm2[...]


def _full(shape):
    return pl.BlockSpec(shape, lambda i: (0,) * len(shape))


def kernel(x, edge_feature, edge_index, node_positions, params):
    f32 = jnp.float32
    n_nodes, d_node = x.shape
    n_edges, d_edge = edge_feature.shape
    row = edge_index[0].astype(jnp.int32)
    col = edge_index[1].astype(jnp.int32)

    keys = row * n_nodes + col
    rkeys = col * n_nodes + row
    order = jnp.argsort(keys)
    skeys = keys[order]
    pos = jnp.searchsorted(skeys, rkeys)
    pos_c = jnp.clip(pos, 0, n_edges - 1)
    found = skeys[pos_c] == rkeys
    gidx = order[pos_c]

    posp = jnp.pad(node_positions, ((0, 0), (0, 1)))
    xi, xj, rev = _sc_gathers(x, edge_feature, row, col, gidx)
    pi = jnp.take(posp, row, axis=0)
    pj = jnp.take(posp, col, axis=0)
    foundf = found.astype(f32)[:, None]

    (w1, b1), (w2, b2) = params['nn_edge_update']
    w1a, w1b, w1c, w1d = w1[:128], w1[128:256], w1[256:384], w1[384:512]
    (wq, bq) = params['proj_q'][0]
    (wk, bk) = params['proj_k'][0]
    (wv, bv) = params['proj_v'][0]
    (wa1, ba1), (wa2, ba2) = params['nn_att']
    bq_ = jnp.zeros((128, 256), f32)
    bk_ = jnp.zeros((128, 256), f32)
    b2_ = jnp.zeros((256, 128), f32)
    for h in range(_H):
        bq_ = bq_.at[16 * h:16 * h + 16, 32 * h:32 * h + 32].set(wa1[:16])
        bk_ = bk_.at[16 * h:16 * h + 16, 32 * h:32 * h + 32].set(wa1[16:])
        b2_ = b2_.at[32 * h:32 * h + 32, 16 * h:16 * h + 16].set(wa2)
    wxa = wq @ bq_
    wea = wk @ bk_
    ba = (bq @ bq_ + bk @ bk_ + jnp.tile(ba1, _H))[None]
    wb2 = b2_
    bb2 = jnp.tile(ba2, _H)[None]
    (wd1, bd1), (wd2, bd2) = params['distance_mlp']

    be_blk = min(1000, n_edges)
    grid_e = n_edges // be_blk
    espec = pl.BlockSpec((be_blk, d_node), lambda i: (i, 0))
    pspec = pl.BlockSpec((be_blk, 4), lambda i: (i, 0))
    fspec = pl.BlockSpec((be_blk, 1), lambda i: (i, 0))
    ue, msg = pl.pallas_call(
        _edge_body,
        grid=(grid_e,),
        in_specs=[espec, espec, espec, espec, pspec, pspec, fspec,
                  _full((128, 384)), _full((128, 384)), _full((128, 384)),
                  _full((128, 384)), _full((1, 384)),
                  _full((384, 128)), _full((1, 128)),
                  _full((128, 256)), _full((128, 256)), _full((1, 256)),
                  _full((256, 128)), _full((1, 128)),
                  _full((128, 128)), _full((1, 128)),
                  _full((4, 32)), _full((1, 32)),
                  _full((32, 1)), _full((1, 1))],
        out_specs=[espec, espec],
        out_shape=[jax.ShapeDtypeStruct((n_edges, d_edge), f32),
                   jax.ShapeDtypeStruct((n_edges, d_node), f32)],
        compiler_params=pltpu.CompilerParams(
            dimension_semantics=("arbitrary",)),
    )(xi, edge_feature, rev, xj, pi, pj, foundf,
      w1a, w1b, w1c, w1d, b1[None], w2, b2[None],
      wxa, wea, ba, wb2, bb2,
      wv, bv[None], wd1, bd1[None], wd2, bd2[None])

    bounds = jnp.searchsorted(skeys, jnp.arange(n_nodes + 1, dtype=jnp.int32) * n_nodes)
    deg_out = (bounds[1:] - bounds[:-1]).astype(f32)
    deg_in = jax.ops.segment_sum(jnp.ones((n_edges,), f32), col, num_segments=n_nodes)
    agg = jax.ops.segment_max(msg, row, num_segments=n_nodes)
    agg = jnp.where(deg_out[:, None] > 0, agg, 0.0)
    s_out = jax.ops.segment_sum(ue, row, num_segments=n_nodes)
    s_in = jax.ops.segment_sum(ue, col, num_segments=n_nodes)

    we_w, we_b = params['edge_attention_mlp'][0]
    weo, wei = we_w[:128], we_w[128:]
    (wn1, bn1), (wn2, bn2) = params['nn_node_update']
    wn1a, wn1b = wn1[:128], wn1[128:]
    (wm1, bm1), (wm2, bm2) = params['node_nonlinear_mlp']

    bn_blk = min(1000, n_nodes)
    grid_n = n_nodes // bn_blk
    nspec = pl.BlockSpec((bn_blk, d_node), lambda i: (i, 0))
    dspec = pl.BlockSpec((bn_blk, 1), lambda i: (i, 0))
    new_node = pl.pallas_call(
        _node_body,
        grid=(grid_n,),
        in_specs=[nspec, nspec, nspec, nspec, dspec, dspec,
                  _full((128, 128)), _full((128, 128)), _full((1, 128)),
                  _full((128, 256)), _full((128, 256)), _full((1, 256)),
                  _full((256, 128)), _full((1, 128)),
                  _full((128, 128)), _full((1, 128)),
                  _full((128, 128)), _full((1, 128))],
        out_specs=nspec,
        out_shape=jax.ShapeDtypeStruct((n_nodes, d_node), f32),
        compiler_params=pltpu.CompilerParams(
            dimension_semantics=("arbitrary",)),
    )(x, agg, s_out, s_in, deg_out[:, None], deg_in[:, None],
      weo, wei, we_b[None], wn1a, wn1b, bn1[None], wn2, bn2[None],
      wm1, bm1[None], wm2, bm2[None])

    return new_node, ue
